# baseline (speedup 1.0000x reference)
import functools

import jax
import jax.numpy as jnp
from jax import lax
from jax.experimental import pallas as pl
from jax.experimental.pallas import tpu as pltpu
from jax.experimental.pallas import tpu_sc as plsc

NOBS = 16384
CH = 10
NCOLS = 33
L = 16

_info = plsc.get_sparse_core_info()
NC = _info.num_cores
NS = _info.num_subcores
NW = NC * NS
BPW = NOBS // NW
FPW = BPW * CH
NCHUNK = BPW // L
NBLEND = FPW // L
GCH = 128
NG = (3 * BPW) // GCH

_mesh = plsc.VectorSubcoreMesh(core_axis_name="c", subcore_axis_name="s")


@functools.partial(
    pl.kernel,
    mesh=_mesh,
    compiler_params=pltpu.CompilerParams(needs_layout_passes=False),
    out_type=[
        jax.ShapeDtypeStruct((NOBS * CH,), jnp.float32),
        jax.ShapeDtypeStruct((NOBS,), jnp.float32),
    ],
    scratch_types=[
        pltpu.VMEM((2 * BPW,), jnp.int32),
        pltpu.VMEM((3 * BPW,), jnp.int32),
        pltpu.VMEM((3 * BPW,), jnp.float32),
        pltpu.VMEM((BPW,), jnp.float32),
        pltpu.VMEM((FPW,), jnp.float32),
        pltpu.VMEM((FPW,), jnp.float32),
        pltpu.VMEM((FPW,), jnp.float32),
        pltpu.SemaphoreType.DMA,
        pltpu.SemaphoreType.DMA,
        pltpu.SemaphoreType.DMA,
    ],
)
def _seaice_sc(geo_hbm, eo_hbm, es_hbm, tab_hbm, out_hbm, s_hbm,
               geo_v, idx_v, g_v, s_v, eo_v, es_v, out_v,
               sem_g, sem_e, sem_s):
    wid = lax.axis_index("s") * NC + lax.axis_index("c")
    base = wid * BPW
    fbase = wid * FPW

    cp_eo = pltpu.async_copy(eo_hbm.at[pl.ds(fbase, FPW)], eo_v, sem_e)
    cp_es = pltpu.async_copy(es_hbm.at[pl.ds(fbase, FPW)], es_v, sem_e)

    pltpu.sync_copy(geo_hbm.at[pl.ds(2 * base, 2 * BPW)], geo_v)

    lane = lax.iota(jnp.int32, L)

    def idx_body(j, carry):
        i16 = j * L + lane
        r = plsc.load_gather(geo_v, [2 * i16])
        c = plsc.load_gather(geo_v, [2 * i16 + 1])
        f = r * NCOLS + c
        off = j * L
        idx_v[pl.ds(off, L)] = f
        idx_v[pl.ds(BPW + off, L)] = f + 1
        idx_v[pl.ds(2 * BPW + off, L)] = f + 2
        return carry

    lax.fori_loop(0, NCHUNK, idx_body, 0)

    gathers = [
        pltpu.async_copy(tab_hbm.at[idx_v.at[pl.ds(t * GCH, GCH)]],
                         g_v.at[pl.ds(t * GCH, GCH)], sem_g)
        for t in range(NG)
    ]
    for cp in gathers:
        cp.wait()

    a0 = jnp.float32(0.2)
    a1 = jnp.float32(0.3)
    a2 = jnp.float32(0.5)

    def s_body(j, carry):
        off = j * L
        g0 = g_v[pl.ds(off, L)]
        g1 = g_v[pl.ds(BPW + off, L)]
        g2 = g_v[pl.ds(2 * BPW + off, L)]
        s_v[pl.ds(off, L)] = a0 * g0 + a1 * g1 + a2 * g2
        return carry

    lax.fori_loop(0, NCHUNK, s_body, 0)

    cp_s = pltpu.async_copy(s_v, s_hbm.at[pl.ds(base, BPW)], sem_s)

    cp_eo.wait()
    cp_es.wait()

    ten = jnp.int32(CH)

    def blend_body(j, carry):
        off = j * L
        p = off + lane
        d = p // ten
        sg = plsc.load_gather(s_v, [d])
        eo = eo_v[pl.ds(off, L)]
        es = es_v[pl.ds(off, L)]
        out_v[pl.ds(off, L)] = eo + sg * (es - eo)
        return carry

    lax.fori_loop(0, NBLEND, blend_body, 0)

    pltpu.sync_copy(out_v, out_hbm.at[pl.ds(fbase, FPW)])
    cp_s.wait()


def kernel(geolocation, emis_ocean, emis_seaice, tsfc, seaice, seaice_background):
    del tsfc, seaice_background
    geo_flat = geolocation.reshape(-1)
    tab_flat = seaice.reshape(-1)
    eo_flat = emis_ocean.reshape(-1)
    es_flat = emis_seaice.reshape(-1)
    out_flat, s = _seaice_sc(geo_flat, eo_flat, es_flat, tab_flat)
    return (out_flat.reshape(NOBS, CH), s)


# speedup vs baseline: 1.0724x; 1.0724x over previous
import functools

import jax
import jax.numpy as jnp
from jax import lax
from jax.experimental import pallas as pl
from jax.experimental.pallas import tpu as pltpu
from jax.experimental.pallas import tpu_sc as plsc

NOBS = 16384
CH = 10
NCOLS = 33
L = 16

_info = plsc.get_sparse_core_info()
NC = _info.num_cores
NS = _info.num_subcores
NW = NC * NS
BPW = NOBS // NW
FPW = BPW * CH
NCHUNK = BPW // L
GCH = 128
NG = (3 * BPW) // GCH
RCH = 64
NR = BPW // RCH

_mesh = plsc.VectorSubcoreMesh(core_axis_name="c", subcore_axis_name="s")


@functools.partial(
    pl.kernel,
    mesh=_mesh,
    compiler_params=pltpu.CompilerParams(needs_layout_passes=False),
    out_type=[
        jax.ShapeDtypeStruct((NOBS, CH), jnp.float32),
        jax.ShapeDtypeStruct((NOBS,), jnp.float32),
    ],
    scratch_types=[
        pltpu.VMEM((RCH, 2), jnp.int32),
        pltpu.VMEM((3 * BPW,), jnp.int32),
        pltpu.VMEM((3 * BPW,), jnp.float32),
        pltpu.VMEM((BPW,), jnp.float32),
        pltpu.VMEM((FPW,), jnp.float32),
        pltpu.VMEM((FPW,), jnp.float32),
        pltpu.VMEM((RCH, CH), jnp.float32),
        pltpu.SemaphoreType.DMA,
        pltpu.SemaphoreType.DMA,
        pltpu.SemaphoreType.DMA,
    ],
)
def _seaice_sc(geo_hbm, eo_hbm, es_hbm, tab_hbm, out_hbm, s_hbm,
               geo_c, idx_v, g_v, s_v, eo_v, es_v, out_c,
               sem_g, sem_e, sem_s):
    wid = lax.axis_index("s") * NC + lax.axis_index("c")
    base = wid * BPW
    fbase = wid * FPW

    cp_eo = pltpu.async_copy(eo_hbm.at[pl.ds(fbase, FPW)], eo_v, sem_e)
    cp_es = pltpu.async_copy(es_hbm.at[pl.ds(fbase, FPW)], es_v, sem_e)

    lane = lax.iota(jnp.int32, L)
    zero16 = lane - lane
    one16 = zero16 + 1

    for g in range(NR):
        pltpu.sync_copy(geo_hbm.at[pl.ds(base + g * RCH, RCH), :], geo_c)

        def idx_body(j, carry, g=g):
            i16 = j * L + lane
            r = plsc.load_gather(geo_c, [i16, zero16])
            c = plsc.load_gather(geo_c, [i16, one16])
            f = r * NCOLS + c
            off = g * RCH + j * L
            idx_v[pl.ds(off, L)] = f
            idx_v[pl.ds(BPW + off, L)] = f + 1
            idx_v[pl.ds(2 * BPW + off, L)] = f + 2
            return carry

        lax.fori_loop(0, RCH // L, idx_body, 0)

    gathers = [
        pltpu.async_copy(tab_hbm.at[idx_v.at[pl.ds(t * GCH, GCH)]],
                         g_v.at[pl.ds(t * GCH, GCH)], sem_g)
        for t in range(NG)
    ]
    for cp in gathers:
        cp.wait()

    a0 = jnp.float32(0.2)
    a1 = jnp.float32(0.3)
    a2 = jnp.float32(0.5)

    def s_body(j, carry):
        off = j * L
        g0 = g_v[pl.ds(off, L)]
        g1 = g_v[pl.ds(BPW + off, L)]
        g2 = g_v[pl.ds(2 * BPW + off, L)]
        s_v[pl.ds(off, L)] = a0 * g0 + a1 * g1 + a2 * g2
        return carry

    lax.fori_loop(0, NCHUNK, s_body, 0)

    cp_s = pltpu.async_copy(s_v, s_hbm.at[pl.ds(base, BPW)], sem_s)

    cp_eo.wait()
    cp_es.wait()

    ten = jnp.int32(CH)

    for g in range(NR):
        def blend_body(j, carry, g=g):
            p = g * RCH * CH + j * L + lane
            d = p // ten
            m = p - d * ten
            sg = plsc.load_gather(s_v, [d])
            off = g * RCH * CH + j * L
            eo = eo_v[pl.ds(off, L)]
            es = es_v[pl.ds(off, L)]
            plsc.store_scatter(out_c, [d - g * RCH, m], eo + sg * (es - eo))
            return carry

        lax.fori_loop(0, (RCH * CH) // L, blend_body, 0)
        pltpu.sync_copy(out_c, out_hbm.at[pl.ds(base + g * RCH, RCH), :])


def kernel(geolocation, emis_ocean, emis_seaice, tsfc, seaice, seaice_background):
    del tsfc, seaice_background
    tab_flat = seaice.reshape(-1)
    eo_flat = emis_ocean.reshape(-1)
    es_flat = emis_seaice.reshape(-1)
    out, s = _seaice_sc(geolocation, eo_flat, es_flat, tab_flat)
    return (out, s)


# speedup vs baseline: 1.0741x; 1.0016x over previous
import functools

import jax
import jax.numpy as jnp
from jax import lax
from jax.experimental import pallas as pl
from jax.experimental.pallas import tpu as pltpu
from jax.experimental.pallas import tpu_sc as plsc

NOBS = 16384
CH = 10
NCOLS = 33
L = 16

_info = plsc.get_sparse_core_info()
NC = _info.num_cores
NS = _info.num_subcores
NW = NC * NS
BPW = NOBS // NW
FPW = BPW * CH
NCHUNK = BPW // L
GCH = 128
NG = (3 * BPW) // GCH
RCH = 64
NR = BPW // RCH

_mesh = plsc.VectorSubcoreMesh(core_axis_name="c", subcore_axis_name="s")


@functools.partial(
    pl.kernel,
    mesh=_mesh,
    compiler_params=pltpu.CompilerParams(
        needs_layout_passes=False, skip_device_barrier=True),
    out_type=[
        jax.ShapeDtypeStruct((NOBS, CH), jnp.float32),
        jax.ShapeDtypeStruct((NOBS,), jnp.float32),
    ],
    scratch_types=[
        pltpu.VMEM((RCH, 2), jnp.int32),
        pltpu.VMEM((3 * BPW,), jnp.int32),
        pltpu.VMEM((3 * BPW,), jnp.float32),
        pltpu.VMEM((BPW,), jnp.float32),
        pltpu.VMEM((FPW,), jnp.float32),
        pltpu.VMEM((FPW,), jnp.float32),
        pltpu.VMEM((RCH, CH), jnp.float32),
        pltpu.SemaphoreType.DMA,
        pltpu.SemaphoreType.DMA,
        pltpu.SemaphoreType.DMA,
    ],
)
def _seaice_sc(geo_hbm, eo_hbm, es_hbm, tab_hbm, out_hbm, s_hbm,
               geo_c, idx_v, g_v, s_v, eo_v, es_v, out_c,
               sem_g, sem_e, sem_s):
    wid = lax.axis_index("s") * NC + lax.axis_index("c")
    base = wid * BPW
    fbase = wid * FPW

    cp_eo = pltpu.async_copy(eo_hbm.at[pl.ds(fbase, FPW)], eo_v, sem_e)
    cp_es = pltpu.async_copy(es_hbm.at[pl.ds(fbase, FPW)], es_v, sem_e)

    lane = lax.iota(jnp.int32, L)
    zero16 = lane - lane
    one16 = zero16 + 1

    for g in range(NR):
        pltpu.sync_copy(geo_hbm.at[pl.ds(base + g * RCH, RCH), :], geo_c)

        def idx_body(j, carry, g=g):
            i16 = j * L + lane
            r = plsc.load_gather(geo_c, [i16, zero16])
            c = plsc.load_gather(geo_c, [i16, one16])
            f = r * NCOLS + c
            off = g * RCH + j * L
            idx_v[pl.ds(off, L)] = f
            idx_v[pl.ds(BPW + off, L)] = f + 1
            idx_v[pl.ds(2 * BPW + off, L)] = f + 2
            return carry

        lax.fori_loop(0, RCH // L, idx_body, 0)

    gathers = [
        pltpu.async_copy(tab_hbm.at[idx_v.at[pl.ds(t * GCH, GCH)]],
                         g_v.at[pl.ds(t * GCH, GCH)], sem_g)
        for t in range(NG)
    ]
    for cp in gathers:
        cp.wait()

    a0 = jnp.float32(0.2)
    a1 = jnp.float32(0.3)
    a2 = jnp.float32(0.5)

    def s_body(j, carry):
        off = j * L
        g0 = g_v[pl.ds(off, L)]
        g1 = g_v[pl.ds(BPW + off, L)]
        g2 = g_v[pl.ds(2 * BPW + off, L)]
        s_v[pl.ds(off, L)] = a0 * g0 + a1 * g1 + a2 * g2
        return carry

    lax.fori_loop(0, NCHUNK, s_body, 0)

    cp_s = pltpu.async_copy(s_v, s_hbm.at[pl.ds(base, BPW)], sem_s)

    cp_eo.wait()
    cp_es.wait()

    ten = jnp.int32(CH)

    for g in range(NR):
        def blend_body(j, carry, g=g):
            p = g * RCH * CH + j * L + lane
            d = p // ten
            m = p - d * ten
            sg = plsc.load_gather(s_v, [d])
            off = g * RCH * CH + j * L
            eo = eo_v[pl.ds(off, L)]
            es = es_v[pl.ds(off, L)]
            plsc.store_scatter(out_c, [d - g * RCH, m], eo + sg * (es - eo))
            return carry

        lax.fori_loop(0, (RCH * CH) // L, blend_body, 0)
        pltpu.sync_copy(out_c, out_hbm.at[pl.ds(base + g * RCH, RCH), :])


def kernel(geolocation, emis_ocean, emis_seaice, tsfc, seaice, seaice_background):
    del tsfc, seaice_background
    tab_flat = seaice.reshape(-1)
    eo_flat = emis_ocean.reshape(-1)
    es_flat = emis_seaice.reshape(-1)
    out, s = _seaice_sc(geolocation, eo_flat, es_flat, tab_flat)
    return (out, s)


# speedup vs baseline: 1.3941x; 1.2979x over previous
import functools

import jax
import jax.numpy as jnp
from jax import lax
from jax.experimental import pallas as pl
from jax.experimental.pallas import tpu as pltpu
from jax.experimental.pallas import tpu_sc as plsc

NOBS = 16384
CH = 10
NCOLS = 33
L = 16

_info = plsc.get_sparse_core_info()
NC = _info.num_cores
NS = _info.num_subcores
NW = NC * NS
BPW = NOBS // NW
RCH = 64
NWAVE = BPW // RCH

_mesh = plsc.VectorSubcoreMesh(core_axis_name="c", subcore_axis_name="s")

_f32 = jnp.float32
_i32 = jnp.int32


@functools.partial(
    pl.kernel,
    mesh=_mesh,
    compiler_params=pltpu.CompilerParams(needs_layout_passes=False),
    out_type=[
        jax.ShapeDtypeStruct((NOBS, CH), _f32),
        jax.ShapeDtypeStruct((NOBS,), _f32),
    ],
    scratch_types=[
        pltpu.VMEM((BPW,), _i32),
        pltpu.VMEM((BPW,), _i32),
        pltpu.VMEM((BPW,), _f32),
        [pltpu.VMEM((RCH, NCOLS), _f32) for _ in range(2)],
        [pltpu.VMEM((RCH, CH), _f32) for _ in range(2)],
        [pltpu.VMEM((RCH, CH), _f32) for _ in range(2)],
        [pltpu.VMEM((RCH, CH), _f32) for _ in range(2)],
        pltpu.VMEM((RCH, 2), _i32),
        [pltpu.SemaphoreType.DMA for _ in range(2)],
        [pltpu.SemaphoreType.DMA for _ in range(2)],
        [pltpu.SemaphoreType.DMA for _ in range(2)],
        [pltpu.SemaphoreType.DMA for _ in range(2)],
        pltpu.SemaphoreType.DMA,
        pltpu.SemaphoreType.DMA,
    ],
)
def _seaice_sc(geo_hbm, eo_hbm, es_hbm, tab_hbm, out_hbm, s_hbm,
               row_v, col_v, s_v, rows_b, eo_b, es_b, out_b, geo_i,
               sem_rows, sem_eo, sem_es, sem_out, sem_g, sem_s):
    wid = lax.axis_index("s") * NC + lax.axis_index("c")
    base = wid * BPW

    lane0 = lax.iota(_i32, L)
    zero16 = lane0 - lane0
    one16 = zero16 + 1

    for g in range(NWAVE):
        pltpu.sync_copy(
            geo_hbm.at[pl.ds(base + g * RCH, RCH), :], geo_i)

        def geo_body(j, carry, g=g):
            i16 = j * L + lane0
            r = plsc.load_gather(geo_i, [i16, zero16])
            c = plsc.load_gather(geo_i, [i16, one16])
            off = g * RCH + j * L
            row_v[pl.ds(off, L)] = r
            col_v[pl.ds(off, L)] = c
            return carry

        lax.fori_loop(0, RCH // L, geo_body, 0)

    def fire_emis(w):
        b = w % 2
        return (
            pltpu.async_copy(
                eo_hbm.at[pl.ds(base + w * RCH, RCH), :], eo_b[b], sem_eo[b]),
            pltpu.async_copy(
                es_hbm.at[pl.ds(base + w * RCH, RCH), :], es_b[b], sem_es[b]),
        )

    def fire_wave(w):
        b = w % 2
        buf = rows_b[b]

        def body(j, carry):
            rv = row_v[pl.ds(w * RCH + j * L, L)]
            for k in range(L):
                pltpu.async_copy(tab_hbm.at[pl.ds(rv[k], 1), :],
                                 buf.at[pl.ds(j * L + k, 1), :],
                                 sem_rows[b])
            return carry

        lax.fori_loop(0, RCH // L, body, 0)

    def drain_wave(w):
        b = w % 2
        pltpu.make_async_copy(
            tab_hbm.at[pl.ds(0, RCH), :], rows_b[b], sem_rows[b]).wait()

    lane = lax.iota(_i32, L)
    a0 = _f32(0.2)
    a1 = _f32(0.3)
    a2 = _f32(0.5)
    ten = _i32(CH)

    cp_emis = [None] * NWAVE
    cp_out = [None] * NWAVE

    cp_emis[0] = fire_emis(0)
    fire_wave(0)

    for w in range(NWAVE):
        if w + 1 < NWAVE:
            fire_wave(w + 1)
            cp_emis[w + 1] = fire_emis(w + 1)

        drain_wave(w)
        buf = rows_b[w % 2]

        def s_body(j, carry, w=w, buf=buf):
            i16 = j * L + lane
            off = w * RCH + j * L
            c = col_v[pl.ds(off, L)]
            g0 = plsc.load_gather(buf, [i16, c])
            g1 = plsc.load_gather(buf, [i16, c + 1])
            g2 = plsc.load_gather(buf, [i16, c + 2])
            s_v[pl.ds(off, L)] = a0 * g0 + a1 * g1 + a2 * g2
            return carry

        lax.fori_loop(0, RCH // L, s_body, 0)

        if w >= 2:
            cp_out[w - 2].wait()
        for cp in cp_emis[w]:
            cp.wait()

        b = w % 2
        eo_c, es_c, out_c = eo_b[b], es_b[b], out_b[b]

        def blend_body(j, carry, w=w, eo_c=eo_c, es_c=es_c, out_c=out_c):
            p = j * L + lane
            d = p // ten
            m = p - d * ten
            sg = plsc.load_gather(s_v, [w * RCH + d])
            eo = plsc.load_gather(eo_c, [d, m])
            es = plsc.load_gather(es_c, [d, m])
            plsc.store_scatter(out_c, [d, m], eo + sg * (es - eo))
            return carry

        lax.fori_loop(0, (RCH * CH) // L, blend_body, 0)

        cp_out[w] = pltpu.async_copy(
            out_c, out_hbm.at[pl.ds(base + w * RCH, RCH), :], sem_out[b])

    cp_s = pltpu.async_copy(s_v, s_hbm.at[pl.ds(base, BPW)], sem_s)
    cp_out[NWAVE - 2].wait()
    cp_out[NWAVE - 1].wait()
    cp_s.wait()


def kernel(geolocation, emis_ocean, emis_seaice, tsfc, seaice, seaice_background):
    del tsfc, seaice_background
    out, s = _seaice_sc(geolocation, emis_ocean, emis_seaice, seaice)
    return (out, s)
